# baseline (speedup 1.0000x reference)
import functools
import jax
import jax.numpy as jnp
from jax import lax
from jax.experimental import pallas as pl
from jax.experimental.pallas import tpu as pltpu
from jax.experimental.pallas import tpu_sc as plsc

N = 16384
B = 64
LANES = 16
NC = 2
NS = 16
NW = NC * NS
IDXW = 128
BLK = 128
SPB = BLK // IDXW


def _sc_spmm(nblocks, xt, cols2d, rows2d, vals):
    mesh = plsc.VectorSubcoreMesh(core_axis_name="c", subcore_axis_name="s")
    rows_per_tile = N // NS

    @functools.partial(
        pl.kernel,
        out_type=jax.ShapeDtypeStruct((NC, N, B), jnp.float32),
        mesh=mesh,
        scratch_types=[
            pltpu.VMEM((nblocks * SPB, IDXW), jnp.int32),
            pltpu.VMEM((nblocks * SPB, IDXW), jnp.int32),
            pltpu.VMEM((nblocks * SPB, IDXW), jnp.float32),
            pltpu.VMEM((BLK, B), jnp.float32),
            pltpu.VMEM((BLK, B), jnp.float32),
            pltpu.VMEM_SHARED((N, B), jnp.float32),
            pltpu.SemaphoreType.DMA,
            pltpu.SemaphoreType.DMA,
        ],
        compiler_params=pltpu.CompilerParams(use_tc_tiling_on_sc=False),
    )
    def kern(xt_hbm, cols_hbm, rows_hbm, vals_hbm, out_hbm,
             cols_v, rows_v, vals_v, gath0, gath1, y_sh, gsem, ssem):
        cid = lax.axis_index("c")
        sid = lax.axis_index("s")
        wid = cid * NS + sid

        @pl.loop(0, BLK)
        def _(k):
            for j in range(B // LANES):
                gath0[k, pl.ds(j * LANES, LANES)] = jnp.zeros((LANES,), jnp.float32)

        for r in range(rows_per_tile // BLK):
            pltpu.sync_copy(gath0, y_sh.at[pl.ds(sid * rows_per_tile + r * BLK, BLK)])
        plsc.subcore_barrier()

        nrows = nblocks * SPB
        pltpu.sync_copy(cols_hbm.at[pl.ds(wid * nrows, nrows)], cols_v)
        pltpu.sync_copy(rows_hbm.at[pl.ds(wid * nrows, nrows)], rows_v)
        pltpu.sync_copy(vals_hbm.at[pl.ds(wid * nrows, nrows)], vals_v)

        def gather_copies(b, buf):
            return [
                pltpu.make_async_copy(
                    xt_hbm.at[cols_v.at[b * SPB + j]],
                    buf.at[pl.ds(j * IDXW, IDXW)],
                    gsem,
                )
                for j in range(SPB)
            ]

        def fire_gathers(b, buf):
            for c in gather_copies(b, buf):
                c.start()

        def wait_gathers(b, buf):
            for c in gather_copies(b, buf):
                c.wait()

        def scale_and_fire(b, buf):
            for j in range(SPB):
                @plsc.parallel_loop(0, IDXW // LANES, unroll=2)
                def _(g):
                    vv16 = vals_v[b * SPB + j, pl.ds(g * LANES, LANES)]
                    for i in range(LANES):
                        vv = jnp.full((LANES,), vv16[i], jnp.float32)
                        k = j * IDXW + g * LANES + i
                        for q in range(B // LANES):
                            sl = (k, pl.ds(q * LANES, LANES))
                            buf[sl] = buf[sl] * vv

                pltpu.async_copy(
                    buf.at[pl.ds(j * IDXW, IDXW)],
                    y_sh.at[rows_v.at[b * SPB + j]],
                    ssem,
                    add=True,
                )

        def drain_scatters(b, buf):
            for j in range(SPB):
                pltpu.make_async_copy(
                    buf.at[pl.ds(j * IDXW, IDXW)],
                    y_sh.at[rows_v.at[b * SPB + j]],
                    ssem,
                ).wait()

        fire_gathers(0, gath0)
        fire_gathers(1, gath1)

        @pl.loop(0, nblocks, step=2)
        def _(b):
            wait_gathers(b, gath0)
            scale_and_fire(b, gath0)
            drain_scatters(b, gath0)

            @pl.when(b + 2 < nblocks)
            def _():
                fire_gathers(b + 2, gath0)

            wait_gathers(b + 1, gath1)
            scale_and_fire(b + 1, gath1)
            drain_scatters(b + 1, gath1)

            @pl.when(b + 3 < nblocks)
            def _():
                fire_gathers(b + 3, gath1)

        plsc.subcore_barrier()
        for r in range(rows_per_tile // BLK):
            base = sid * rows_per_tile + r * BLK
            pltpu.sync_copy(y_sh.at[pl.ds(base, BLK)], out_hbm.at[cid].at[pl.ds(base, BLK)])

    return kern(xt, cols2d, rows2d, vals)


_TBLK = 2048


def _tc_transpose(x):
    def body(x_ref, o_ref):
        o_ref[...] = x_ref[...].T

    return pl.pallas_call(
        body,
        grid=(N // _TBLK,),
        in_specs=[pl.BlockSpec((B, _TBLK), lambda i: (0, i))],
        out_specs=pl.BlockSpec((_TBLK, B), lambda i: (i, 0)),
        out_shape=jax.ShapeDtypeStruct((N, B), jnp.float32),
    )(x)


def _tc_combine(o, bias2d):
    def body(o_ref, b_ref, y_ref):
        s = o_ref[0] + o_ref[1]
        y_ref[...] = s.T + b_ref[...]

    return pl.pallas_call(
        body,
        grid=(N // _TBLK,),
        in_specs=[
            pl.BlockSpec((2, _TBLK, B), lambda i: (0, i, 0)),
            pl.BlockSpec((1, _TBLK), lambda i: (0, i)),
        ],
        out_specs=pl.BlockSpec((B, _TBLK), lambda i: (0, i)),
        out_shape=jax.ShapeDtypeStruct((B, N), jnp.float32),
    )(o, bias2d)


@jax.jit
def kernel(x, st_indices, st_values, bias):
    nnz = st_values.shape[0]
    per_tile = -(-nnz // (NW * 2 * BLK)) * 2 * BLK
    nblocks = per_tile // BLK
    pad = NW * per_tile

    rows = st_indices[0]
    cols = st_indices[1]
    tail = jnp.arange(nnz, pad, dtype=jnp.int32) % N
    cols_p = jnp.concatenate([cols, tail])
    rows_p = jnp.concatenate([rows, tail])
    vals_p = jnp.concatenate([st_values, jnp.zeros((pad - nnz,), jnp.float32)])

    xt = _tc_transpose(x)
    out = _sc_spmm(
        nblocks,
        xt,
        cols_p.reshape(-1, IDXW),
        rows_p.reshape(-1, IDXW),
        vals_p.reshape(-1, IDXW),
    )
    return _tc_combine(out, bias.reshape(1, N))


# speedup vs baseline: 1.0457x; 1.0457x over previous
import functools
import jax
import jax.numpy as jnp
from jax import lax
from jax.experimental import pallas as pl
from jax.experimental.pallas import tpu as pltpu
from jax.experimental.pallas import tpu_sc as plsc

N = 16384
B = 64
LANES = 16
NC = 2
NS = 16
NW = NC * NS
IDXW = 128
BLK = 256
SPB = BLK // IDXW


def _sc_spmm(nblocks, xt, cols2d, rows2d, vals):
    mesh = plsc.VectorSubcoreMesh(core_axis_name="c", subcore_axis_name="s")
    rows_per_tile = N // NS

    @functools.partial(
        pl.kernel,
        out_type=jax.ShapeDtypeStruct((NC, N, B), jnp.float32),
        mesh=mesh,
        scratch_types=[
            pltpu.VMEM((nblocks * SPB, IDXW), jnp.int32),
            pltpu.VMEM((nblocks * SPB, IDXW), jnp.int32),
            pltpu.VMEM((nblocks * SPB, IDXW), jnp.float32),
            pltpu.VMEM((BLK, B), jnp.float32),
            pltpu.VMEM((BLK, B), jnp.float32),
            pltpu.VMEM_SHARED((N, B), jnp.float32),
            pltpu.SemaphoreType.DMA,
            pltpu.SemaphoreType.DMA,
        ],
        compiler_params=pltpu.CompilerParams(use_tc_tiling_on_sc=False),
    )
    def kern(xt_hbm, cols_hbm, rows_hbm, vals_hbm, out_hbm,
             cols_v, rows_v, vals_v, gath0, gath1, y_sh, gsem, ssem):
        cid = lax.axis_index("c")
        sid = lax.axis_index("s")
        wid = cid * NS + sid

        @pl.loop(0, BLK)
        def _(k):
            for j in range(B // LANES):
                gath0[k, pl.ds(j * LANES, LANES)] = jnp.zeros((LANES,), jnp.float32)

        for r in range(rows_per_tile // BLK):
            pltpu.sync_copy(gath0, y_sh.at[pl.ds(sid * rows_per_tile + r * BLK, BLK)])
        plsc.subcore_barrier()

        nrows = nblocks * SPB
        pltpu.sync_copy(cols_hbm.at[pl.ds(wid * nrows, nrows)], cols_v)
        pltpu.sync_copy(rows_hbm.at[pl.ds(wid * nrows, nrows)], rows_v)
        pltpu.sync_copy(vals_hbm.at[pl.ds(wid * nrows, nrows)], vals_v)

        def gather_copies(b, buf):
            return [
                pltpu.make_async_copy(
                    xt_hbm.at[cols_v.at[b * SPB + j]],
                    buf.at[pl.ds(j * IDXW, IDXW)],
                    gsem,
                )
                for j in range(SPB)
            ]

        def fire_gathers(b, buf):
            for c in gather_copies(b, buf):
                c.start()

        def wait_gathers(b, buf):
            for c in gather_copies(b, buf):
                c.wait()

        def scale_and_fire(b, buf):
            for j in range(SPB):
                @plsc.parallel_loop(0, IDXW // LANES, unroll=2)
                def _(g):
                    vv16 = vals_v[b * SPB + j, pl.ds(g * LANES, LANES)]
                    for i in range(LANES):
                        vv = jnp.full((LANES,), vv16[i], jnp.float32)
                        k = j * IDXW + g * LANES + i
                        for q in range(B // LANES):
                            sl = (k, pl.ds(q * LANES, LANES))
                            buf[sl] = buf[sl] * vv

                pltpu.async_copy(
                    buf.at[pl.ds(j * IDXW, IDXW)],
                    y_sh.at[rows_v.at[b * SPB + j]],
                    ssem,
                    add=True,
                )

        def drain_scatters(b, buf):
            for j in range(SPB):
                pltpu.make_async_copy(
                    buf.at[pl.ds(j * IDXW, IDXW)],
                    y_sh.at[rows_v.at[b * SPB + j]],
                    ssem,
                ).wait()

        fire_gathers(0, gath0)
        fire_gathers(1, gath1)

        @pl.loop(0, nblocks, step=2)
        def _(b):
            wait_gathers(b, gath0)
            scale_and_fire(b, gath0)
            drain_scatters(b, gath0)

            @pl.when(b + 2 < nblocks)
            def _():
                fire_gathers(b + 2, gath0)

            wait_gathers(b + 1, gath1)
            scale_and_fire(b + 1, gath1)
            drain_scatters(b + 1, gath1)

            @pl.when(b + 3 < nblocks)
            def _():
                fire_gathers(b + 3, gath1)

        plsc.subcore_barrier()
        for r in range(rows_per_tile // BLK):
            base = sid * rows_per_tile + r * BLK
            pltpu.sync_copy(y_sh.at[pl.ds(base, BLK)], out_hbm.at[cid].at[pl.ds(base, BLK)])

    return kern(xt, cols2d, rows2d, vals)


_TBLK = 2048


def _tc_transpose(x):
    def body(x_ref, o_ref):
        o_ref[...] = x_ref[...].T

    return pl.pallas_call(
        body,
        grid=(N // _TBLK,),
        in_specs=[pl.BlockSpec((B, _TBLK), lambda i: (0, i))],
        out_specs=pl.BlockSpec((_TBLK, B), lambda i: (i, 0)),
        out_shape=jax.ShapeDtypeStruct((N, B), jnp.float32),
    )(x)


def _tc_combine(o, bias2d):
    def body(o_ref, b_ref, y_ref):
        s = o_ref[0] + o_ref[1]
        y_ref[...] = s.T + b_ref[...]

    return pl.pallas_call(
        body,
        grid=(N // _TBLK,),
        in_specs=[
            pl.BlockSpec((2, _TBLK, B), lambda i: (0, i, 0)),
            pl.BlockSpec((1, _TBLK), lambda i: (0, i)),
        ],
        out_specs=pl.BlockSpec((B, _TBLK), lambda i: (0, i)),
        out_shape=jax.ShapeDtypeStruct((B, N), jnp.float32),
    )(o, bias2d)


@jax.jit
def kernel(x, st_indices, st_values, bias):
    nnz = st_values.shape[0]
    per_tile = -(-nnz // (NW * 2 * BLK)) * 2 * BLK
    nblocks = per_tile // BLK
    pad = NW * per_tile

    rows = st_indices[0]
    cols = st_indices[1]
    tail = jnp.arange(nnz, pad, dtype=jnp.int32) % N
    cols_p = jnp.concatenate([cols, tail])
    rows_p = jnp.concatenate([rows, tail])
    vals_p = jnp.concatenate([st_values, jnp.zeros((pad - nnz,), jnp.float32)])

    xt = _tc_transpose(x)
    out = _sc_spmm(
        nblocks,
        xt,
        cols_p.reshape(-1, IDXW),
        rows_p.reshape(-1, IDXW),
        vals_p.reshape(-1, IDXW),
    )
    return _tc_combine(out, bias.reshape(1, N))


# speedup vs baseline: 1.0767x; 1.0297x over previous
import functools
import jax
import jax.numpy as jnp
from jax import lax
from jax.experimental import pallas as pl
from jax.experimental.pallas import tpu as pltpu
from jax.experimental.pallas import tpu_sc as plsc

N = 16384
B = 64
LANES = 16
NC = 2
NS = 16
NW = NC * NS
IDXW = 128
BLK = 256
SPB = BLK // IDXW


def _sc_spmm(nblocks, xt, cols2d, rows2d, vals):
    mesh = plsc.VectorSubcoreMesh(core_axis_name="c", subcore_axis_name="s")
    rows_per_tile = N // NS

    @functools.partial(
        pl.kernel,
        out_type=jax.ShapeDtypeStruct((NC, N, B), jnp.float32),
        mesh=mesh,
        scratch_types=[
            pltpu.VMEM((nblocks * SPB, IDXW), jnp.int32),
            pltpu.VMEM((nblocks * SPB, IDXW), jnp.int32),
            pltpu.VMEM((nblocks * SPB, IDXW), jnp.float32),
            pltpu.VMEM((BLK, B), jnp.float32),
            pltpu.VMEM((BLK, B), jnp.float32),
            pltpu.VMEM_SHARED((N, B), jnp.float32),
            pltpu.SemaphoreType.DMA,
            pltpu.SemaphoreType.DMA,
        ],
        compiler_params=pltpu.CompilerParams(use_tc_tiling_on_sc=False),
    )
    def kern(xt_hbm, cols_hbm, rows_hbm, vals_hbm, out_hbm,
             cols_v, rows_v, vals_v, gath0, gath1, y_sh, gsem, ssem):
        cid = lax.axis_index("c")
        sid = lax.axis_index("s")
        wid = cid * NS + sid

        @pl.loop(0, BLK)
        def _(k):
            for j in range(B // LANES):
                gath0[k, pl.ds(j * LANES, LANES)] = jnp.zeros((LANES,), jnp.float32)

        for r in range(rows_per_tile // BLK):
            pltpu.sync_copy(gath0, y_sh.at[pl.ds(sid * rows_per_tile + r * BLK, BLK)])
        plsc.subcore_barrier()

        nrows = nblocks * SPB
        pltpu.sync_copy(cols_hbm.at[pl.ds(wid * nrows, nrows)], cols_v)
        pltpu.sync_copy(rows_hbm.at[pl.ds(wid * nrows, nrows)], rows_v)
        pltpu.sync_copy(vals_hbm.at[pl.ds(wid * nrows, nrows)], vals_v)

        def gather_copies(b, buf):
            return [
                pltpu.make_async_copy(
                    xt_hbm.at[cols_v.at[b * SPB + j]],
                    buf.at[pl.ds(j * IDXW, IDXW)],
                    gsem,
                )
                for j in range(SPB)
            ]

        def fire_gathers(b, buf):
            for c in gather_copies(b, buf):
                c.start()

        def wait_gathers(b, buf):
            for c in gather_copies(b, buf):
                c.wait()

        def scale_and_fire(b, buf):
            for j in range(SPB):
                @plsc.parallel_loop(0, IDXW // LANES, unroll=2)
                def _(g):
                    vv16 = vals_v[b * SPB + j, pl.ds(g * LANES, LANES)]
                    for i in range(LANES):
                        vv = jnp.full((LANES,), vv16[i], jnp.float32)
                        k = j * IDXW + g * LANES + i
                        for q in range(B // LANES):
                            sl = (k, pl.ds(q * LANES, LANES))
                            buf[sl] = buf[sl] * vv

                pltpu.async_copy(
                    buf.at[pl.ds(j * IDXW, IDXW)],
                    y_sh.at[rows_v.at[b * SPB + j]],
                    ssem,
                    add=True,
                )

        def drain_scatters(b, buf):
            for j in range(SPB):
                pltpu.make_async_copy(
                    buf.at[pl.ds(j * IDXW, IDXW)],
                    y_sh.at[rows_v.at[b * SPB + j]],
                    ssem,
                ).wait()

        fire_gathers(0, gath0)
        fire_gathers(1, gath1)

        @pl.loop(0, nblocks, step=2)
        def _(b):
            wait_gathers(b, gath0)
            scale_and_fire(b, gath0)
            drain_scatters(b, gath0)

            @pl.when(b + 2 < nblocks)
            def _():
                fire_gathers(b + 2, gath0)

            wait_gathers(b + 1, gath1)
            scale_and_fire(b + 1, gath1)
            drain_scatters(b + 1, gath1)

            @pl.when(b + 3 < nblocks)
            def _():
                fire_gathers(b + 3, gath1)

        plsc.subcore_barrier()
        for r in range(rows_per_tile // BLK):
            base = sid * rows_per_tile + r * BLK
            pltpu.sync_copy(y_sh.at[pl.ds(base, BLK)], out_hbm.at[cid].at[pl.ds(base, BLK)])

    return kern(xt, cols2d, rows2d, vals)


_TBLK = 4096


def _tc_transpose(x):
    def body(x_ref, o_ref):
        o_ref[...] = x_ref[...].T

    return pl.pallas_call(
        body,
        grid=(N // _TBLK,),
        in_specs=[pl.BlockSpec((B, _TBLK), lambda i: (0, i))],
        out_specs=pl.BlockSpec((_TBLK, B), lambda i: (i, 0)),
        out_shape=jax.ShapeDtypeStruct((N, B), jnp.float32),
    )(x)


def _tc_combine(o, bias2d):
    def body(o_ref, b_ref, y_ref):
        s = o_ref[0] + o_ref[1]
        y_ref[...] = s.T + b_ref[...]

    return pl.pallas_call(
        body,
        grid=(N // _TBLK,),
        in_specs=[
            pl.BlockSpec((2, _TBLK, B), lambda i: (0, i, 0)),
            pl.BlockSpec((1, _TBLK), lambda i: (0, i)),
        ],
        out_specs=pl.BlockSpec((B, _TBLK), lambda i: (0, i)),
        out_shape=jax.ShapeDtypeStruct((B, N), jnp.float32),
    )(o, bias2d)


@jax.jit
def kernel(x, st_indices, st_values, bias):
    nnz = st_values.shape[0]
    per_tile = -(-nnz // (NW * 2 * BLK)) * 2 * BLK
    nblocks = per_tile // BLK
    pad = NW * per_tile

    rows = st_indices[0]
    cols = st_indices[1]
    tail = jnp.arange(nnz, pad, dtype=jnp.int32) % N
    cols_p = jnp.concatenate([cols, tail])
    rows_p = jnp.concatenate([rows, tail])
    vals_p = jnp.concatenate([st_values, jnp.zeros((pad - nnz,), jnp.float32)])

    xt = _tc_transpose(x)
    out = _sc_spmm(
        nblocks,
        xt,
        cols_p.reshape(-1, IDXW),
        rows_p.reshape(-1, IDXW),
        vals_p.reshape(-1, IDXW),
    )
    return _tc_combine(out, bias.reshape(1, N))


# speedup vs baseline: 1.0859x; 1.0085x over previous
import functools
import jax
import jax.numpy as jnp
from jax import lax
from jax.experimental import pallas as pl
from jax.experimental.pallas import tpu as pltpu
from jax.experimental.pallas import tpu_sc as plsc

N = 16384
B = 64
LANES = 16
NC = 2
NS = 16
NW = NC * NS
IDXW = 128
BLK = 256
SPB = BLK // IDXW


def _sc_spmm(nblocks, xt, cols2d, rows2d, vals):
    mesh = plsc.VectorSubcoreMesh(core_axis_name="c", subcore_axis_name="s")
    rows_per_tile = N // NS

    @functools.partial(
        pl.kernel,
        out_type=jax.ShapeDtypeStruct((NC, N, B), jnp.float32),
        mesh=mesh,
        scratch_types=[
            pltpu.VMEM((nblocks * SPB, IDXW), jnp.int32),
            pltpu.VMEM((nblocks * SPB, IDXW), jnp.int32),
            pltpu.VMEM((nblocks * SPB, IDXW), jnp.float32),
            pltpu.VMEM((BLK, B), jnp.float32),
            pltpu.VMEM((BLK, B), jnp.float32),
            pltpu.VMEM_SHARED((N, B), jnp.float32),
            pltpu.SemaphoreType.DMA,
            pltpu.SemaphoreType.DMA,
        ],
        compiler_params=pltpu.CompilerParams(use_tc_tiling_on_sc=False),
    )
    def kern(xt_hbm, cols_hbm, rows_hbm, vals_hbm, out_hbm,
             cols_v, rows_v, vals_v, gath0, gath1, y_sh, gsem, ssem):
        cid = lax.axis_index("c")
        sid = lax.axis_index("s")
        wid = cid * NS + sid

        @pl.loop(0, BLK)
        def _(k):
            for j in range(B // LANES):
                gath0[k, pl.ds(j * LANES, LANES)] = jnp.zeros((LANES,), jnp.float32)

        for r in range(rows_per_tile // BLK):
            pltpu.sync_copy(gath0, y_sh.at[pl.ds(sid * rows_per_tile + r * BLK, BLK)])
        plsc.subcore_barrier()

        nrows = nblocks * SPB
        pltpu.sync_copy(cols_hbm.at[pl.ds(wid * nrows, nrows)], cols_v)
        pltpu.sync_copy(rows_hbm.at[pl.ds(wid * nrows, nrows)], rows_v)
        pltpu.sync_copy(vals_hbm.at[pl.ds(wid * nrows, nrows)], vals_v)

        def gather_copies(b, buf):
            return [
                pltpu.make_async_copy(
                    xt_hbm.at[cols_v.at[b * SPB + j]],
                    buf.at[pl.ds(j * IDXW, IDXW)],
                    gsem,
                )
                for j in range(SPB)
            ]

        def fire_gathers(b, buf):
            for c in gather_copies(b, buf):
                c.start()

        def wait_gathers(b, buf):
            for c in gather_copies(b, buf):
                c.wait()

        def scale_and_fire(b, buf):
            for j in range(SPB):
                @plsc.parallel_loop(0, IDXW // LANES, unroll=2)
                def _(g):
                    vv16 = vals_v[b * SPB + j, pl.ds(g * LANES, LANES)]
                    for i in range(LANES):
                        vv = jnp.full((LANES,), vv16[i], jnp.float32)
                        k = j * IDXW + g * LANES + i
                        for q in range(B // LANES):
                            sl = (k, pl.ds(q * LANES, LANES))
                            buf[sl] = buf[sl] * vv

                pltpu.async_copy(
                    buf.at[pl.ds(j * IDXW, IDXW)],
                    y_sh.at[rows_v.at[b * SPB + j]],
                    ssem,
                    add=True,
                )

        def drain_scatters(b, buf):
            for j in range(SPB):
                pltpu.make_async_copy(
                    buf.at[pl.ds(j * IDXW, IDXW)],
                    y_sh.at[rows_v.at[b * SPB + j]],
                    ssem,
                ).wait()

        fire_gathers(0, gath0)
        fire_gathers(1, gath1)

        @pl.loop(0, nblocks, step=2)
        def _(b):
            wait_gathers(b, gath0)
            scale_and_fire(b, gath0)
            drain_scatters(b, gath0)

            @pl.when(b + 2 < nblocks)
            def _():
                fire_gathers(b + 2, gath0)

            wait_gathers(b + 1, gath1)
            scale_and_fire(b + 1, gath1)
            drain_scatters(b + 1, gath1)

            @pl.when(b + 3 < nblocks)
            def _():
                fire_gathers(b + 3, gath1)

        plsc.subcore_barrier()
        for r in range(rows_per_tile // BLK):
            base = sid * rows_per_tile + r * BLK
            pltpu.sync_copy(y_sh.at[pl.ds(base, BLK)], out_hbm.at[cid].at[pl.ds(base, BLK)])

    return kern(xt, cols2d, rows2d, vals)


_TBLK = 8192


def _tc_transpose(x):
    def body(x_ref, o_ref):
        o_ref[...] = x_ref[...].T

    return pl.pallas_call(
        body,
        grid=(N // _TBLK,),
        in_specs=[pl.BlockSpec((B, _TBLK), lambda i: (0, i))],
        out_specs=pl.BlockSpec((_TBLK, B), lambda i: (i, 0)),
        out_shape=jax.ShapeDtypeStruct((N, B), jnp.float32),
    )(x)


def _tc_combine(o, bias2d):
    def body(o_ref, b_ref, y_ref):
        s = o_ref[0] + o_ref[1]
        y_ref[...] = s.T + b_ref[...]

    return pl.pallas_call(
        body,
        grid=(N // _TBLK,),
        in_specs=[
            pl.BlockSpec((2, _TBLK, B), lambda i: (0, i, 0)),
            pl.BlockSpec((1, _TBLK), lambda i: (0, i)),
        ],
        out_specs=pl.BlockSpec((B, _TBLK), lambda i: (0, i)),
        out_shape=jax.ShapeDtypeStruct((B, N), jnp.float32),
    )(o, bias2d)


@jax.jit
def kernel(x, st_indices, st_values, bias):
    nnz = st_values.shape[0]
    per_tile = -(-nnz // (NW * 2 * BLK)) * 2 * BLK
    nblocks = per_tile // BLK
    pad = NW * per_tile

    rows = st_indices[0]
    cols = st_indices[1]
    tail = jnp.arange(nnz, pad, dtype=jnp.int32) % N
    cols_p = jnp.concatenate([cols, tail])
    rows_p = jnp.concatenate([rows, tail])
    vals_p = jnp.concatenate([st_values, jnp.zeros((pad - nnz,), jnp.float32)])

    xt = _tc_transpose(x)
    out = _sc_spmm(
        nblocks,
        xt,
        cols_p.reshape(-1, IDXW),
        rows_p.reshape(-1, IDXW),
        vals_p.reshape(-1, IDXW),
    )
    return _tc_combine(out, bias.reshape(1, N))
